# baseline (speedup 1.0000x reference)
import math

import numpy as np
import jax
import jax.numpy as jnp
from jax import lax
from jax.experimental import pallas as pl
from jax.experimental.pallas import tpu as pltpu
from jax.experimental.pallas import tpu_sc as plsc

_D = 1024
_B = 4
_S = 4096
_NC = 2
_NS = 16
_NW = _NC * _NS
_WPB = _NW // _B
_TPW = _S // _WPB
_CH = 32
_NCHUNK = _TPW // _CH
_SCALE = math.sqrt(_D)
_LANES = 16
_VPR = _D // _LANES
_NGRP = 3


def _pe_rows(pos: np.ndarray) -> np.ndarray:
    omega = np.power(10000.0, -2.0 * np.arange(_D // 2, dtype=np.float64) / _D)
    angle = pos.astype(np.float64)[:, None] * omega[None, :]
    rows = np.empty((pos.shape[0], _D), dtype=np.float64)
    rows[:, 0::2] = np.sin(angle)
    rows[:, 1::2] = np.cos(angle)
    return rows


def _pe_consts() -> tuple[np.ndarray, np.ndarray]:
    omega = np.power(10000.0, -2.0 * np.arange(_D // 2, dtype=np.float64) / _D)
    cdsd = np.empty((2, _D), dtype=np.float64)
    cdsd[0, 0::2] = cdsd[0, 1::2] = np.cos(omega)
    cdsd[1, 0::2] = np.sin(omega)
    cdsd[1, 1::2] = -np.sin(omega)
    pos = (np.arange(_NW, dtype=np.float64) % _WPB) * _TPW - 1.0
    p = _pe_rows(pos)
    seeds = np.empty((2 * _NW, _D), dtype=np.float64)
    seeds[0::2] = p
    seeds[1::2, 0::2] = p[:, 1::2]
    seeds[1::2, 1::2] = p[:, 0::2]
    return cdsd.astype(np.float32), seeds.astype(np.float32)


_CDSD, _SEEDS = _pe_consts()


def _compute(buf, carry, cdsd):
    @plsc.parallel_loop(0, _VPR, unroll=2)
    def _vecs(j):
        sl = pl.ds(j * _LANES, _LANES)
        cd = cdsd[0, sl]
        sd = cdsd[1, sl]
        p0 = carry[0, sl]
        q0 = carry[1, sl]

        @pl.loop(0, _CH // 8, init_carry=(p0, q0))
        def _rows(r8, pq):
            p, q = pq
            for k in range(8):
                p, q = p * cd + q * sd, q * cd - p * sd
                r = r8 * 8 + k
                buf[r, sl] = buf[r, sl] * _SCALE + p
            return p, q

        p1, q1 = _rows
        carry[0, sl] = p1
        carry[1, sl] = q1


def _body(table, seqs, cdsd_hbm, seeds_hbm, out, idx_v, cdsd_v, carry, *rest):
    bufs = rest[:_NGRP]
    gsems = rest[_NGRP:2 * _NGRP]
    ssems = rest[2 * _NGRP:]

    wid = lax.axis_index("s") * _NC + lax.axis_index("c")
    bat = wid // _WPB
    pos0 = (wid % _WPB) * _TPW

    pltpu.sync_copy(seqs.at[bat, pl.ds(pos0, _TPW)], idx_v)
    pltpu.sync_copy(cdsd_hbm, cdsd_v)
    pltpu.sync_copy(seeds_hbm.at[pl.ds(2 * wid, 2)], carry)

    def start_gather(i):
        g = i % _NGRP
        return pltpu.async_copy(
            table.at[idx_v.at[pl.ds(i * _CH, _CH)]], bufs[g], gsems[g])

    def start_store(i):
        g = i % _NGRP
        return pltpu.async_copy(
            bufs[g], out.at[bat, pl.ds(pos0 + i * _CH, _CH)], ssems[g])

    gd = {0: start_gather(0)}
    sd = {}
    for i in range(_NCHUNK):
        g = i % _NGRP
        if i + 1 < _NCHUNK:
            if i + 1 >= _NGRP:
                sd.pop(i + 1 - _NGRP).wait()
            gd[i + 1] = start_gather(i + 1)
        gd.pop(i).wait()
        _compute(bufs[g], carry, cdsd_v)
        sd[i] = start_store(i)
    for i in sorted(sd):
        sd[i].wait()


def _embed(seqs, table, cdsd, seeds):
    k = pl.kernel(
        _body,
        out_type=jax.ShapeDtypeStruct((_B, _S, _D), jnp.float32),
        mesh=plsc.VectorSubcoreMesh(core_axis_name="c", subcore_axis_name="s"),
        scratch_types=[
            pltpu.VMEM((_TPW,), jnp.int32),
            pltpu.VMEM((2, _D), jnp.float32),
            pltpu.VMEM((2, _D), jnp.float32),
        ]
        + [pltpu.VMEM((_CH, _D), jnp.float32) for _ in range(_NGRP)]
        + [pltpu.SemaphoreType.DMA for _ in range(2 * _NGRP)],
    )
    return k(table, seqs, cdsd, seeds)


def kernel(seqs, embed_weight):
    cdsd = jnp.asarray(_CDSD)
    seeds = jnp.asarray(_SEEDS)
    return jax.jit(_embed)(seqs, embed_weight, cdsd, seeds)


# speedup vs baseline: 1.5687x; 1.5687x over previous
import math

import numpy as np
import jax
import jax.numpy as jnp
from jax import lax
from jax.experimental import pallas as pl
from jax.experimental.pallas import tpu as pltpu
from jax.experimental.pallas import tpu_sc as plsc

_D = 1024
_B = 4
_S = 4096
_NC = 2
_NS = 16
_NW = _NC * _NS
_WPB = _NW // _B
_TPW = _S // _WPB
_CH = 32
_NCHUNK = _TPW // _CH
_SCALE = math.sqrt(_D)
_LANES = 16
_VPR = _D // _LANES
_NGRP = 3


def _pe_rows(pos) -> np.ndarray:
    omega = np.power(10000.0, -2.0 * np.arange(_D // 2, dtype=np.float64) / _D)
    angle = np.asarray(pos, dtype=np.float64)[:, None] * omega[None, :]
    rows = np.empty((len(pos), _D), dtype=np.float64)
    rows[:, 0::2] = np.sin(angle)
    rows[:, 1::2] = np.cos(angle)
    return rows


def _pe_consts() -> tuple[np.ndarray, np.ndarray]:
    omega = np.power(10000.0, -2.0 * np.arange(_D // 2, dtype=np.float64) / _D)
    c2 = np.empty((1, _D), dtype=np.float64)
    c2[0, 0::2] = c2[0, 1::2] = 2.0 * np.cos(omega)
    seeds = np.empty((2 * _NW, _D), dtype=np.float64)
    for w in range(_NW):
        start = (w % _WPB) * _TPW
        seeds[2 * w:2 * w + 2] = _pe_rows([start - 2.0, start - 1.0])
    return c2.astype(np.float32), seeds.astype(np.float32)


_C2, _SEEDS = _pe_consts()


def _compute(buf, carry, c2_v):
    @plsc.parallel_loop(0, _VPR)
    def _vecs(j):
        sl = pl.ds(j * _LANES, _LANES)
        c2 = c2_v[0, sl]
        pm1 = carry[0, sl]
        p = carry[1, sl]
        for k in range(_CH):
            pn = c2 * p - pm1
            buf[k, sl] = buf[k, sl] * _SCALE + pn
            pm1, p = p, pn
        carry[0, sl] = pm1
        carry[1, sl] = p


def _body(table, seqs, c2_hbm, seeds_hbm, out, idx_v, c2_v, carry, *rest):
    bufs = rest[:_NGRP]
    gsems = rest[_NGRP:2 * _NGRP]
    ssems = rest[2 * _NGRP:]

    wid = lax.axis_index("s") * _NC + lax.axis_index("c")
    bat = wid // _WPB
    pos0 = (wid % _WPB) * _TPW

    pltpu.sync_copy(seqs.at[bat, pl.ds(pos0, _TPW)], idx_v)
    pltpu.sync_copy(c2_hbm, c2_v)
    pltpu.sync_copy(seeds_hbm.at[pl.ds(2 * wid, 2)], carry)

    def start_gather(i):
        g = i % _NGRP
        return pltpu.async_copy(
            table.at[idx_v.at[pl.ds(i * _CH, _CH)]], bufs[g], gsems[g])

    def start_store(i):
        g = i % _NGRP
        return pltpu.async_copy(
            bufs[g], out.at[bat, pl.ds(pos0 + i * _CH, _CH)], ssems[g])

    gd = {0: start_gather(0)}
    sd = {}
    for i in range(_NCHUNK):
        g = i % _NGRP
        if i + 1 < _NCHUNK:
            if i + 1 >= _NGRP:
                sd.pop(i + 1 - _NGRP).wait()
            gd[i + 1] = start_gather(i + 1)
        gd.pop(i).wait()
        _compute(bufs[g], carry, c2_v)
        sd[i] = start_store(i)
    for i in sorted(sd):
        sd[i].wait()


def _embed(seqs, table, c2, seeds):
    k = pl.kernel(
        _body,
        out_type=jax.ShapeDtypeStruct((_B, _S, _D), jnp.float32),
        mesh=plsc.VectorSubcoreMesh(core_axis_name="c", subcore_axis_name="s"),
        scratch_types=[
            pltpu.VMEM((_TPW,), jnp.int32),
            pltpu.VMEM((1, _D), jnp.float32),
            pltpu.VMEM((2, _D), jnp.float32),
        ]
        + [pltpu.VMEM((_CH, _D), jnp.float32) for _ in range(_NGRP)]
        + [pltpu.SemaphoreType.DMA for _ in range(2 * _NGRP)],
    )
    return k(table, seqs, c2, seeds)


def kernel(seqs, embed_weight):
    c2 = jnp.asarray(_C2)
    seeds = jnp.asarray(_SEEDS)
    return jax.jit(_embed)(seqs, embed_weight, c2, seeds)


# speedup vs baseline: 1.6109x; 1.0269x over previous
import math

import numpy as np
import jax
import jax.numpy as jnp
from jax import lax
from jax.experimental import pallas as pl
from jax.experimental.pallas import tpu as pltpu
from jax.experimental.pallas import tpu_sc as plsc

_D = 1024
_B = 4
_S = 4096
_NC = 2
_NS = 16
_NW = _NC * _NS
_WPB = _NW // _B
_TPW = _S // _WPB
_CH = 32
_NCHUNK = _TPW // _CH
_SCALE = math.sqrt(_D)
_LANES = 16
_VPR = _D // _LANES
_NGRP = 3


def _pe_rows(pos) -> np.ndarray:
    omega = np.power(10000.0, -2.0 * np.arange(_D // 2, dtype=np.float64) / _D)
    angle = np.asarray(pos, dtype=np.float64)[:, None] * omega[None, :]
    rows = np.empty((len(pos), _D), dtype=np.float64)
    rows[:, 0::2] = np.sin(angle)
    rows[:, 1::2] = np.cos(angle)
    return rows


_NCHAIN = 4


def _pe_consts() -> tuple[np.ndarray, np.ndarray]:
    omega = np.power(10000.0, -2.0 * np.arange(_D // 2, dtype=np.float64) / _D)
    c2 = np.empty((1, _D), dtype=np.float64)
    c2[0, 0::2] = c2[0, 1::2] = 2.0 * np.cos(_NCHAIN * omega)
    seeds = np.empty((2 * _NCHAIN * _NW, _D), dtype=np.float64)
    for w in range(_NW):
        start = (w % _WPB) * _TPW
        seeds[2 * _NCHAIN * w:2 * _NCHAIN * (w + 1)] = _pe_rows(
            np.arange(start - 2.0 * _NCHAIN, start))
    return c2.astype(np.float32), seeds.astype(np.float32)


_C2, _SEEDS = _pe_consts()


def _compute(buf, carry, c2_v):
    @plsc.parallel_loop(0, _VPR)
    def _vecs(j):
        sl = pl.ds(j * _LANES, _LANES)
        c2 = c2_v[0, sl]
        st = [carry[r, sl] for r in range(2 * _NCHAIN)]
        for k in range(_CH):
            pn = c2 * st[_NCHAIN] - st[0]
            buf[k, sl] = buf[k, sl] * _SCALE + pn
            st = st[1:] + [pn]
        for r in range(2 * _NCHAIN):
            carry[r, sl] = st[r]


def _body(table, seqs, c2_hbm, seeds_hbm, out, idx_v, c2_v, carry, *rest):
    bufs = rest[:_NGRP]
    gsems = rest[_NGRP:2 * _NGRP]
    ssems = rest[2 * _NGRP:]

    wid = lax.axis_index("s") * _NC + lax.axis_index("c")
    bat = wid // _WPB
    pos0 = (wid % _WPB) * _TPW

    pltpu.sync_copy(seqs.at[bat, pl.ds(pos0, _TPW)], idx_v)
    pltpu.sync_copy(c2_hbm, c2_v)
    pltpu.sync_copy(seeds_hbm.at[pl.ds(2 * _NCHAIN * wid, 2 * _NCHAIN)], carry)

    def start_gather(i):
        g = i % _NGRP
        return pltpu.async_copy(
            table.at[idx_v.at[pl.ds(i * _CH, _CH)]], bufs[g], gsems[g])

    def start_store(i):
        g = i % _NGRP
        return pltpu.async_copy(
            bufs[g], out.at[bat, pl.ds(pos0 + i * _CH, _CH)], ssems[g])

    gd = {0: start_gather(0)}
    sd = {}
    for i in range(_NCHUNK):
        g = i % _NGRP
        if i + 1 < _NCHUNK:
            if i + 1 >= _NGRP:
                sd.pop(i + 1 - _NGRP).wait()
            gd[i + 1] = start_gather(i + 1)
        gd.pop(i).wait()
        _compute(bufs[g], carry, c2_v)
        sd[i] = start_store(i)
    for i in sorted(sd):
        sd[i].wait()


def _embed(seqs, table, c2, seeds):
    k = pl.kernel(
        _body,
        out_type=jax.ShapeDtypeStruct((_B, _S, _D), jnp.float32),
        mesh=plsc.VectorSubcoreMesh(core_axis_name="c", subcore_axis_name="s"),
        scratch_types=[
            pltpu.VMEM((_TPW,), jnp.int32),
            pltpu.VMEM((1, _D), jnp.float32),
            pltpu.VMEM((2 * _NCHAIN, _D), jnp.float32),
        ]
        + [pltpu.VMEM((_CH, _D), jnp.float32) for _ in range(_NGRP)]
        + [pltpu.SemaphoreType.DMA for _ in range(2 * _NGRP)],
    )
    return k(table, seqs, c2, seeds)


def kernel(seqs, embed_weight):
    c2 = jnp.asarray(_C2)
    seeds = jnp.asarray(_SEEDS)
    return jax.jit(_embed)(seqs, embed_weight, c2, seeds)
